# baseline (speedup 1.0000x reference)
import jax
import jax.numpy as jnp
from jax.experimental import pallas as pl
from jax.experimental.pallas import tpu as pltpu
from jax.experimental.pallas import tpu_sc as plsc

_B, _S = 4096, 50
_D = 256
_N = _B * _S
_W = 128


def _sc_gather(table, idx2d):
  vector_mesh = plsc.VectorSubcoreMesh(
      core_axis_name="core", subcore_axis_name="subcore"
  )

  @pl.kernel(
      out_type=jax.ShapeDtypeStruct((_N, _D), table.dtype),
      mesh=vector_mesh,
  )
  def kernel(table_hbm, i_hbm, o_hbm):
    def body(i_vmem, o_vmem):
      pltpu.sync_copy(table_hbm.at[i_vmem.at[0]], o_vmem)

    pltpu.emit_pipeline(
        body,
        grid=(_N // _W,),
        in_specs=[pl.BlockSpec((1, _W), index_map=lambda i: (0, i))],
        out_specs=[pl.BlockSpec((_W, _D), index_map=lambda i: (i, 0))],
        core_axis_name=("core", "subcore"),
        dimension_semantics=(pltpu.PARALLEL,),
    )(i_hbm, o_hbm)

  return kernel(table, idx2d)


@jax.jit
def kernel(x, table):
  idx2d = x.astype(jnp.int32).reshape(1, _N)
  out = _sc_gather(table, idx2d)
  return out.reshape(_B, _S, _D)


# speedup vs baseline: 4.5402x; 4.5402x over previous
import jax
import jax.numpy as jnp
from jax.experimental import pallas as pl
from jax.experimental.pallas import tpu as pltpu
from jax.experimental.pallas import tpu_sc as plsc

_B, _S = 4096, 50
_D = 256
_N = _B * _S
_W = 128


def _sc_lookup(table, idx2d):
  vector_mesh = plsc.VectorSubcoreMesh(
      core_axis_name="core", subcore_axis_name="subcore"
  )

  @pl.kernel(
      out_type=jax.ShapeDtypeStruct((_N, _D), table.dtype),
      mesh=vector_mesh,
      scratch_types=[pltpu.VMEM((3, _D), jnp.float32)],
  )
  def kernel(table_hbm, i_hbm, o_hbm, tab_vmem):
    pltpu.sync_copy(table_hbm, tab_vmem)

    def body(i_vmem, o_vmem):
      @pl.loop(0, _W, step=16)
      def _(t0):
        tv = i_vmem[0, pl.ds(t0, 16)]
        for k in range(16):
          row = tv[k]
          for g in range(_D // 16):
            o_vmem[t0 + k, pl.ds(g * 16, 16)] = tab_vmem[row, pl.ds(g * 16, 16)]

    pltpu.emit_pipeline(
        body,
        grid=(_N // _W,),
        in_specs=[pl.BlockSpec((1, _W), index_map=lambda i: (0, i))],
        out_specs=[pl.BlockSpec((_W, _D), index_map=lambda i: (i, 0))],
        core_axis_name=("core", "subcore"),
        dimension_semantics=(pltpu.PARALLEL,),
    )(i_hbm, o_hbm)

  return kernel(table, idx2d)


@jax.jit
def kernel(x, table):
  idx2d = x.astype(jnp.int32).reshape(1, _N)
  out = _sc_lookup(table, idx2d)
  return out.reshape(_B, _S, _D)
